# Initial kernel scaffold; baseline (speedup 1.0000x reference)
import jax
import jax.numpy as jnp
from jax.experimental import pallas as pl


def kernel(x):
    raise NotImplementedError("write your pallas kernel here")


# speedup vs baseline: 8.2218x; 8.2218x over previous
import jax
import jax.numpy as jnp
from jax.experimental import pallas as pl

TIME_STEP = 0.001
T = 1000
B_BLK = 32
T_BLK = 200


def _one_hot_kernel(x_ref, out_ref):
    t_idx = pl.program_id(1)
    bins = (x_ref[...] / TIME_STEP).astype(jnp.int32) - t_idx * T_BLK
    iota = jax.lax.broadcasted_iota(jnp.int32, (B_BLK, T_BLK, x_ref.shape[1]), 1)
    out_ref[...] = (bins[:, None, :] == iota).astype(jnp.float32)


def kernel(x):
    B, C = x.shape
    return pl.pallas_call(
        _one_hot_kernel,
        grid=(B // B_BLK, T // T_BLK),
        in_specs=[pl.BlockSpec((B_BLK, C), lambda b, t: (b, 0))],
        out_specs=pl.BlockSpec((B_BLK, T_BLK, C), lambda b, t: (b, t, 0)),
        out_shape=jax.ShapeDtypeStruct((B, T, C), jnp.float32),
    )(x)
